# baseline (speedup 1.0000x reference)
import functools

import jax
import jax.numpy as jnp
from jax import lax
from jax.experimental import pallas as pl
from jax.experimental.pallas import tpu as pltpu
from jax.experimental.pallas import tpu_sc as plsc

B = 16384
RANK = 32
NW = 32
BPW = B // NW
NCH = BPW // 128
GROUPS = BPW // 16

_mesh = plsc.VectorSubcoreMesh(core_axis_name="c", subcore_axis_name="s")


@functools.partial(
    pl.kernel,
    mesh=_mesh,
    out_type=jax.ShapeDtypeStruct((B,), jnp.float32),
    scratch_types=[
        pltpu.VMEM((NCH, 128), jnp.int32),
        pltpu.VMEM((NCH, 128), jnp.int32),
        pltpu.VMEM((BPW, RANK), jnp.float32),
        pltpu.VMEM((BPW, RANK), jnp.float32),
        pltpu.VMEM((BPW,), jnp.float32),
        pltpu.SemaphoreType.DMA,
    ],
    compiler_params=pltpu.CompilerParams(
        needs_layout_passes=False, use_tc_tiling_on_sc=False),
)
def _mf_kernel(user_hbm, anime_hbm, u_hbm, v_hbm, out_hbm,
               uidx, aidx, u_rows, v_rows, out_v, sem):
    wid = lax.axis_index("s") * 2 + lax.axis_index("c")

    pltpu.sync_copy(user_hbm.at[pl.ds(wid * NCH, NCH)], uidx)
    pltpu.sync_copy(anime_hbm.at[pl.ds(wid * NCH, NCH)], aidx)

    copies = []
    for k in range(NCH):
        copies.append(pltpu.async_copy(
            u_hbm.at[uidx.at[k]], u_rows.at[pl.ds(k * 128, 128)], sem))
        copies.append(pltpu.async_copy(
            v_hbm.at[aidx.at[k]], v_rows.at[pl.ds(k * 128, 128)], sem))
    for c in copies:
        c.wait()

    lane = lax.iota(jnp.int32, 16)

    def group_body(g, carry):
        row = g * 16 + lane
        acc = jnp.zeros((16,), jnp.float32)
        for j in range(RANK):
            col = jnp.full((16,), j, jnp.int32)
            uu = plsc.load_gather(u_rows, [row, col])
            vv = plsc.load_gather(v_rows, [row, col])
            acc = acc + uu * vv
        out_v[pl.ds(g * 16, 16)] = acc
        return carry

    lax.fori_loop(0, GROUPS, group_body, 0)

    pltpu.sync_copy(out_v, out_hbm.at[pl.ds(wid * BPW, BPW)])


def kernel(user, anime, U, V):
    user = user.astype(jnp.int32).reshape(NW * NCH, 128)
    anime = anime.astype(jnp.int32).reshape(NW * NCH, 128)
    return _mf_kernel(user, anime, U, V)
